# baseline (speedup 1.0000x reference)
import jax
import jax.numpy as jnp
from jax import lax
from jax.experimental import pallas as pl
from jax.experimental.pallas import tpu as pltpu
from jax.experimental.pallas import tpu_sc as plsc

_NUM_CLASSES = 1000
_BATCH = 16384
_NC = 2
_NS = 16
_L = 16
_NW = _NC * _NS
_COLS_PER_W = _BATCH // _NW
_CHUNK = 40
_NCHUNK = _NUM_CLASSES // _CHUNK


def _onehot_t_body(c_hbm, out_hbm, idx_v, buf0, buf1, sem0, sem1):
    wid = lax.axis_index("s") * _NC + lax.axis_index("c")
    col0 = wid * _COLS_PER_W

    pltpu.sync_copy(c_hbm.at[pl.ds(col0, _COLS_PER_W)], idx_v)

    zeros16 = jnp.zeros((_L,), jnp.float32)
    ones16 = jnp.ones((_L,), jnp.float32)
    iota16 = lax.iota(jnp.int32, _L)

    def _zero_fill(buf, r0, r1):
        def body(row, _):
            for off in range(0, _COLS_PER_W, _L):
                buf[row, pl.ds(off, _L)] = zeros16
            return 0

        lax.fori_loop(r0, r1, body, 0)

    def _scan_chunk(buf, span, prev_span):
        def body(i, _):
            off = pl.multiple_of(i * _L, _L)
            cls = idx_v[pl.ds(off, _L)]
            lcol = iota16 + off
            if prev_span is not None:
                p0, p1 = prev_span
                maskp = (cls >= p0) & (cls < p1)
                lrowp = jnp.where(maskp, cls - p0, 0)
                plsc.store_scatter(buf, [lrowp, lcol], zeros16, mask=maskp)
            c0, c1 = span
            mask = (cls >= c0) & (cls < c1)
            lrow = jnp.where(mask, cls - c0, 0)
            plsc.store_scatter(buf, [lrow, lcol], ones16, mask=mask)
            return 0

        lax.fori_loop(0, _COLS_PER_W // _L, body, 0)

    bufs = (buf0, buf1)
    sems = (sem0, sem1)

    def _dst(c0):
        return out_hbm.at[pl.ds(c0, _CHUNK), pl.ds(col0, _COLS_PER_W)]

    for b in range(2):
        c0 = b * _CHUNK
        _zero_fill(bufs[b], 0, _CHUNK)
        _scan_chunk(bufs[b], (c0, c0 + _CHUNK), None)
        pltpu.async_copy(bufs[b], _dst(c0), sems[b])

    def _pair(i, _):
        for b in range(2):
            c0 = pl.multiple_of((2 * i + 2 + b) * _CHUNK, 8)
            p0 = c0 - 2 * _CHUNK
            pltpu.make_async_copy(bufs[b], _dst(c0), sems[b]).wait()
            _scan_chunk(bufs[b], (c0, c0 + _CHUNK), (p0, p0 + _CHUNK))
            pltpu.async_copy(bufs[b], _dst(c0), sems[b])
        return 0

    lax.fori_loop(0, (_NCHUNK - 3) // 2, _pair, 0)

    c0 = (_NCHUNK - 1) * _CHUNK
    p0 = c0 - 2 * _CHUNK
    pltpu.make_async_copy(bufs[0], _dst(c0), sems[0]).wait()
    _scan_chunk(bufs[0], (c0, c0 + _CHUNK), (p0, p0 + _CHUNK))
    pltpu.async_copy(bufs[0], _dst(c0), sems[0])
    pltpu.make_async_copy(bufs[1], _dst(c0 - _CHUNK), sems[1]).wait()
    pltpu.make_async_copy(bufs[0], _dst(c0), sems[0]).wait()


@jax.jit
def kernel(c):
    c = c.astype(jnp.int32)
    mesh = plsc.VectorSubcoreMesh(core_axis_name="c", subcore_axis_name="s")
    run = pl.kernel(
        _onehot_t_body,
        out_type=jax.ShapeDtypeStruct((_NUM_CLASSES, _BATCH), jnp.float32),
        mesh=mesh,
        scratch_types=[
            pltpu.VMEM((_COLS_PER_W,), jnp.int32),
            pltpu.VMEM((_CHUNK, _COLS_PER_W), jnp.float32),
            pltpu.VMEM((_CHUNK, _COLS_PER_W), jnp.float32),
            pltpu.SemaphoreType.DMA,
            pltpu.SemaphoreType.DMA,
        ],
        compiler_params=pltpu.CompilerParams(needs_layout_passes=False),
    )
    return run(c).T
